# Initial kernel scaffold; baseline (speedup 1.0000x reference)
import jax
import jax.numpy as jnp
from jax.experimental import pallas as pl


def kernel(x, edge_index, W1, b1, W2, b2, W3, b3):
    raise NotImplementedError("write your pallas kernel here")


# speedup vs baseline: 6.7770x; 6.7770x over previous
import functools

import jax
import jax.numpy as jnp
from jax import lax
from jax.experimental import pallas as pl
from jax.experimental.pallas import tpu as pltpu
from jax.experimental.pallas import tpu_sc as plsc

N = 10000
E = 320000
NC, NS = 2, 16
NW = NC * NS
C = 128
NCH = 80
ISB = 8
EPW_PAD = NCH * C
E_PAD = EPW_PAD * NW
N_ACC = 10112
STRIPE = N_ACC // NS
LSTR = 624
DUMMY = 10008

_MESH = functools.partial(
    plsc.VectorSubcoreMesh,
    core_axis_name="c", subcore_axis_name="s", num_cores=NC, num_subcores=NS,
)


def _zero_vmem(buf, rows, f):
    @pl.loop(0, rows)
    def _(r):
        @pl.loop(0, f // 16)
        def _(k):
            buf[r, pl.ds(k * 16, 16)] = jnp.zeros((16,), jnp.float32)


def _zero_stripe(acc_sh, zbuf, base):
    nfull = STRIPE // C
    tail = STRIPE - nfull * C
    for t in range(nfull):
        pltpu.sync_copy(zbuf, acc_sh.at[pl.ds(base + t * C, C)])
    if tail:
        pltpu.sync_copy(zbuf.at[pl.ds(0, tail)],
                        acc_sh.at[pl.ds(base + nfull * C, tail)])


def _copy_out(acc_sh, out_hbm, c, s):
    pltpu.sync_copy(acc_sh.at[pl.ds(s * LSTR, LSTR)],
                    out_hbm.at[c, pl.ds(s * LSTR, LSTR)])

    @pl.when(s == 0)
    def _():
        pltpu.sync_copy(acc_sh.at[pl.ds(NS * LSTR, N - NS * LSTR)],
                        out_hbm.at[c, pl.ds(NS * LSTR, N - NS * LSTR)])


def _sc_degree():
    @functools.partial(
        pl.kernel,
        out_type=jax.ShapeDtypeStruct((NC, N, 128), jnp.float32),
        mesh=_MESH(),
        scratch_types=[
            pltpu.VMEM((NCH, C), jnp.int32),
            pltpu.VMEM((C, 128), jnp.float32),
            pltpu.VMEM_SHARED((N_ACC, 128), jnp.float32),
        ],
    )
    def k(dstr_hbm, out_hbm, idv, ones_v, acc_sh):
        c = lax.axis_index("c")
        s = lax.axis_index("s")
        wid = s * NC + c
        pltpu.sync_copy(dstr_hbm.at[wid], idv)
        _zero_vmem(ones_v, C, 128)
        _zero_stripe(acc_sh, ones_v, s * STRIPE)

        @pl.loop(0, C)
        def _(r):
            @pl.loop(0, 8)
            def _(k2):
                ones_v[r, pl.ds(k2 * 16, 16)] = jnp.ones((16,), jnp.float32)

        plsc.subcore_barrier()

        @pl.loop(0, NCH)
        def _(j):
            pltpu.sync_copy(ones_v, acc_sh.at[idv.at[j]], add=True)

        plsc.subcore_barrier()
        _copy_out(acc_sh, out_hbm, c, s)

    return k


def _sc_edge_sum128():
    @functools.partial(
        pl.kernel,
        out_type=jax.ShapeDtypeStruct((NC, N, 128), jnp.float32),
        mesh=_MESH(),
        scratch_types=[
            pltpu.VMEM((ISB, C), jnp.int32),
            pltpu.VMEM((NCH, C), jnp.int32),
            pltpu.VMEM((C, 128), jnp.float32),
            pltpu.VMEM_SHARED((N_ACC, 128), jnp.float32),
            pltpu.SemaphoreType.DMA,
        ],
    )
    def k(tab_hbm, srcr_hbm, dstr_hbm, out_hbm,
          isv, idv, rows0, acc_sh, sem0):
        c = lax.axis_index("c")
        s = lax.axis_index("s")
        wid = s * NC + c
        pltpu.sync_copy(dstr_hbm.at[wid], idv)
        _zero_vmem(rows0, C, 128)
        _zero_stripe(acc_sh, rows0, s * STRIPE)
        plsc.subcore_barrier()

        @pl.loop(0, NCH // ISB)
        def _(g):
            pltpu.sync_copy(srcr_hbm.at[wid, pl.ds(g * ISB, ISB)], isv)
            for jj in range(ISB):
                j = g * ISB + jj
                pltpu.async_copy(tab_hbm.at[isv.at[jj]], rows0, sem0).wait()
                pltpu.sync_copy(rows0, acc_sh.at[idv.at[j]], add=True)

        plsc.subcore_barrier()
        _copy_out(acc_sh, out_hbm, c, s)

    return k


def _dot(a, b):
    return jnp.dot(a, b, preferred_element_type=jnp.float32,
                   precision=lax.Precision.HIGHEST)


def _pad128(v):
    return jnp.pad(v, ((0, 0), (0, 128 - v.shape[1])))


def _scale_body(deg_ref, x_ref, dis_ref, xp_ref):
    d = deg_ref[0, :, 0:1] + deg_ref[1, :, 0:1] + 1.0
    dis = lax.rsqrt(d)
    dis_ref[...] = dis
    xp_ref[...] = x_ref[...] * dis


def _layer1_body(acc_ref, xp_ref, dis_ref, w1_ref, b1_ref, w2_ref, o_ref):
    dis = dis_ref[...]
    t = dis * (acc_ref[0] + acc_ref[1] + xp_ref[...])
    h1 = jnp.maximum(_dot(t, w1_ref[...]) + b1_ref[...][None, :], 0.0)
    o_ref[...] = _pad128(dis * _dot(h1, w2_ref[...]))


def _layer2_body(acc_ref, hp_ref, dis_ref, b_ref, o_ref):
    dis = dis_ref[...]
    t = acc_ref[0, :, :32] + acc_ref[1, :, :32] + hp_ref[:, :32]
    h = jnp.maximum(dis * t + b_ref[...][None, :], 0.0)
    o_ref[...] = _pad128(dis * h)


def _out_body(acc_ref, g_ref, dis_ref, w_ref, b_ref, o_ref):
    dis = dis_ref[...]
    t = dis * (acc_ref[0, :, :32] + acc_ref[1, :, :32] + g_ref[:, :32])
    o_ref[...] = _dot(t, w_ref[...]) + b_ref[...][None, :]


def _tc(body, out_shape, *args):
    return pl.pallas_call(
        body, out_shape=jax.ShapeDtypeStruct(out_shape, jnp.float32))(*args)


def kernel(x, edge_index, W1, b1, W2, b2, W3, b3):
    src = edge_index[0].astype(jnp.int32)
    dst = edge_index[1].astype(jnp.int32)
    pad = E_PAD - E
    src_r = jnp.concatenate(
        [src, jnp.zeros((pad,), jnp.int32)]).reshape(NW, NCH, C)
    dst_r = jnp.concatenate(
        [dst, jnp.full((pad,), DUMMY, jnp.int32)]).reshape(NW, NCH, C)

    W3p = jnp.pad(W3, ((0, 0), (0, 126)))
    b3p = jnp.pad(b3, (0, 126))

    deg2 = _sc_degree()(dst_r)
    dis, xp = pl.pallas_call(
        _scale_body,
        out_shape=(jax.ShapeDtypeStruct((N, 1), jnp.float32),
                   jax.ShapeDtypeStruct((N, 128), jnp.float32)),
    )(deg2, x)

    seg = _sc_edge_sum128()
    acc1 = seg(xp, src_r, dst_r)
    hp2p = _tc(_layer1_body, (N, 128), acc1, xp, dis, W1, b1, W2)
    acc2 = seg(hp2p, src_r, dst_r)
    g2p = _tc(_layer2_body, (N, 128), acc2, hp2p, dis, b2)
    acc3 = seg(g2p, src_r, dst_r)
    outp = _tc(_out_body, (N, 128), acc3, g2p, dis, W3p, b3p)
    return outp[:, :2]
